# baseline (speedup 1.0000x reference)
import jax
import jax.numpy as jnp
from jax.experimental import pallas as pl
from jax.experimental.pallas import tpu as pltpu

_B, _N, _D = 16, 64, 128
_GB = 4


def _main_kernel(wlins_ref, wmlp_ref, x_ref, out_ref, wb_s, ws_s, wg_s):
    @pl.when(pl.program_id(0) == 0)
    def _fold():
        W = wlins_ref[...]
        M = wmlp_ref[...]

        def f(k):
            return jnp.dot(W[k - 1], M, preferred_element_type=jnp.float32,
                           precision=jax.lax.Precision.HIGHEST)

        wb_s[...] = jnp.concatenate([f(10), f(11)], axis=1)
        row_r = jnp.concatenate([f(6), f(7), f(3)], axis=1)
        row_c = jnp.concatenate([f(8), f(9), f(4)], axis=1)
        row_d = jnp.concatenate([f(12), f(13), f(1)], axis=1)
        ws_s[...] = jnp.concatenate([row_r, row_c, row_d], axis=0)
        z = jnp.zeros((_D, _D), jnp.float32)
        top = jnp.concatenate([2.0 * f(14), f(2)], axis=1)
        bot = jnp.concatenate([z, f(5)], axis=1)
        wg_s[...] = jnp.concatenate([top, bot], axis=0)

    x = x_ref[...]
    wb = wb_s[...]
    ws = ws_s[...]
    wg = wg_s[...]

    r = jnp.mean(x, axis=2)
    c = jnp.mean(x, axis=1)
    i8 = jax.lax.broadcasted_iota(jnp.int32, (8, 8), 0)
    j8 = jax.lax.broadcasted_iota(jnp.int32, (8, 8), 1)
    eye8 = (i8 == j8).astype(jnp.float32)[None, :, :, None]
    dd = jnp.concatenate(
        [jnp.sum(x[:, 8 * t:8 * t + 8, 8 * t:8 * t + 8, :] * eye8, axis=2)
         for t in range(_N // 8)], axis=1)

    F = jnp.concatenate([r, c, dd], axis=-1)
    S = jax.lax.dot_general(F.reshape(_GB * _N, 3 * _D), ws,
                            (((1,), (0,)), ((), ())),
                            preferred_element_type=jnp.float32)
    S = S.reshape(_GB, _N, 3 * _D)

    s = jnp.mean(dd, axis=1)
    g = jnp.mean(r, axis=1)
    Sg = jnp.concatenate([s, g], axis=-1)
    consts = jnp.sum(Sg[:, :, None] * wg[None, :, :], axis=1)

    x3 = x.reshape(_GB * _N * _N, _D).astype(jnp.bfloat16)
    wbb = wb.astype(jnp.bfloat16)
    Y = jnp.dot(x3, wbb[:, :_D],
                preferred_element_type=jnp.float32).reshape(_GB, _N, _N, _D)
    Zb = jnp.dot(x3, wbb[:, _D:], preferred_element_type=jnp.float32
                 ).astype(jnp.bfloat16).reshape(_GB, _N, _N, _D)
    acc = Y + jnp.swapaxes(Zb, 1, 2).astype(jnp.float32)
    acc += S[:, :, None, :_D]
    colk = S[:, :, _D:2 * _D] + consts[:, None, :_D]
    acc += colk[:, None, :, :]
    out_ref[...] = jnp.maximum(acc, 0.0)
    diag = S[..., 2 * _D:] + consts[:, None, _D:]
    for t in range(_N // 8):
        sl = slice(8 * t, 8 * t + 8)
        blk = acc[:, sl, sl, :] + eye8 * diag[:, sl, None, :]
        out_ref[:, sl, sl, :] = jnp.maximum(blk, 0.0)


def kernel(A, X, Wlins, Wmlp):
    del A
    out = pl.pallas_call(
        _main_kernel,
        grid=(_B // _GB,),
        in_specs=[
            pl.BlockSpec((15, _D, _D), lambda b: (0, 0, 0)),
            pl.BlockSpec((_D, _D), lambda b: (0, 0)),
            pl.BlockSpec((_GB, _N, _N, _D), lambda b: (b, 0, 0, 0)),
        ],
        out_specs=pl.BlockSpec((_GB, _N, _N, _D), lambda b: (b, 0, 0, 0)),
        out_shape=jax.ShapeDtypeStruct((_B, _N, _N, _D), jnp.float32),
        scratch_shapes=[
            pltpu.VMEM((_D, 2 * _D), jnp.float32),
            pltpu.VMEM((3 * _D, 3 * _D), jnp.float32),
            pltpu.VMEM((2 * _D, 2 * _D), jnp.float32),
        ],
    )(Wlins, Wmlp, X)
    return out


# speedup vs baseline: 1.0188x; 1.0188x over previous
import jax
import jax.numpy as jnp
from jax.experimental import pallas as pl
from jax.experimental.pallas import tpu as pltpu

_B, _N, _D = 16, 64, 128
_GB = 4


def _main_kernel(wlins_ref, wmlp_ref, x_ref, out_ref, wb_s, ws_s, wg_s):
    @pl.when(pl.program_id(0) == 0)
    def _fold():
        W = wlins_ref[...]
        M = wmlp_ref[...]

        def f(k):
            return jnp.dot(W[k - 1], M, preferred_element_type=jnp.float32,
                           precision=jax.lax.Precision.HIGHEST)

        wb_s[...] = jnp.concatenate([f(10), f(11)], axis=1)
        row_r = jnp.concatenate([f(6), f(7), f(3)], axis=1)
        row_c = jnp.concatenate([f(8), f(9), f(4)], axis=1)
        row_d = jnp.concatenate([f(12), f(13), f(1)], axis=1)
        ws_s[...] = jnp.concatenate([row_r, row_c, row_d], axis=0)
        z = jnp.zeros((_D, _D), jnp.float32)
        top = jnp.concatenate([2.0 * f(14), f(2)], axis=1)
        bot = jnp.concatenate([z, f(5)], axis=1)
        wg_s[...] = jnp.concatenate([top, bot], axis=0)

    x = x_ref[...]
    wb = wb_s[...]
    ws = ws_s[...]
    wg = wg_s[...]

    r = jnp.mean(x, axis=2)
    c = jnp.mean(x, axis=1)
    i8 = jax.lax.broadcasted_iota(jnp.int32, (8, 8), 0)
    j8 = jax.lax.broadcasted_iota(jnp.int32, (8, 8), 1)
    eye8 = (i8 == j8).astype(jnp.float32)[None, :, :, None]
    dd = jnp.concatenate(
        [jnp.sum(x[:, 8 * t:8 * t + 8, 8 * t:8 * t + 8, :] * eye8, axis=2)
         for t in range(_N // 8)], axis=1)

    F = jnp.concatenate([r, c, dd], axis=-1)
    S = jax.lax.dot_general(F.reshape(_GB * _N, 3 * _D), ws,
                            (((1,), (0,)), ((), ())),
                            preferred_element_type=jnp.float32)
    S = S.reshape(_GB, _N, 3 * _D)

    s = jnp.mean(dd, axis=1)
    g = jnp.mean(r, axis=1)
    Sg = jnp.concatenate([s, g], axis=-1)
    consts = jnp.sum(Sg[:, :, None] * wg[None, :, :], axis=1)

    x3 = x.reshape(_GB * _N * _N, _D)
    Y = jnp.dot(x3, wb[:, :_D],
                preferred_element_type=jnp.float32).reshape(_GB, _N, _N, _D)
    Zb = jnp.dot(x3, wb[:, _D:], preferred_element_type=jnp.float32
                 ).astype(jnp.bfloat16).reshape(_GB, _N, _N, _D)
    rowt = S[..., :_D]
    colk = S[:, :, _D:2 * _D] + consts[:, None, :_D]
    out_ref[...] = jnp.maximum(
        Y + jnp.swapaxes(Zb, 1, 2).astype(jnp.float32)
        + rowt[:, :, None, :] + colk[:, None, :, :], 0.0)
    diag = S[..., 2 * _D:] + consts[:, None, _D:]
    for t in range(_N // 8):
        sl = slice(8 * t, 8 * t + 8)
        blk = (Y[:, sl, sl, :]
               + jnp.swapaxes(Zb[:, sl, sl, :], 1, 2).astype(jnp.float32)
               + rowt[:, sl, None, :] + colk[:, None, sl, :]
               + eye8 * diag[:, sl, None, :])
        out_ref[:, sl, sl, :] = jnp.maximum(blk, 0.0)


def kernel(A, X, Wlins, Wmlp):
    del A
    out = pl.pallas_call(
        _main_kernel,
        grid=(_B // _GB,),
        in_specs=[
            pl.BlockSpec((15, _D, _D), lambda b: (0, 0, 0)),
            pl.BlockSpec((_D, _D), lambda b: (0, 0)),
            pl.BlockSpec((_GB, _N, _N, _D), lambda b: (b, 0, 0, 0)),
        ],
        out_specs=pl.BlockSpec((_GB, _N, _N, _D), lambda b: (b, 0, 0, 0)),
        out_shape=jax.ShapeDtypeStruct((_B, _N, _N, _D), jnp.float32),
        scratch_shapes=[
            pltpu.VMEM((_D, 2 * _D), jnp.float32),
            pltpu.VMEM((3 * _D, 3 * _D), jnp.float32),
            pltpu.VMEM((2 * _D, 2 * _D), jnp.float32),
        ],
    )(Wlins, Wmlp, X)
    return out


# speedup vs baseline: 1.0407x; 1.0215x over previous
import jax
import jax.numpy as jnp
from jax.experimental import pallas as pl
from jax.experimental.pallas import tpu as pltpu

_B, _N, _D = 16, 64, 128
_GB = 4


def _main_kernel(wlins_ref, wmlp_ref, x_ref, out_ref, wb_s, ws_s, wg_s):
    @pl.when(pl.program_id(0) == 0)
    def _fold():
        W = wlins_ref[...]
        M = wmlp_ref[...]

        def f(k):
            return jnp.dot(W[k - 1], M, preferred_element_type=jnp.float32,
                           precision=jax.lax.Precision.HIGHEST)

        wb_s[...] = jnp.concatenate([f(10), f(11)], axis=1)
        row_r = jnp.concatenate([f(6), f(7), f(3)], axis=1)
        row_c = jnp.concatenate([f(8), f(9), f(4)], axis=1)
        row_d = jnp.concatenate([f(12), f(13), f(1)], axis=1)
        ws_s[...] = jnp.concatenate([row_r, row_c, row_d], axis=0)
        z = jnp.zeros((_D, _D), jnp.float32)
        top = jnp.concatenate([2.0 * f(14), f(2)], axis=1)
        bot = jnp.concatenate([z, f(5)], axis=1)
        wg_s[...] = jnp.concatenate([top, bot], axis=0)

    x = x_ref[...]
    wb = wb_s[...]
    ws = ws_s[...]
    wg = wg_s[...]

    r = jnp.mean(x, axis=2)
    c = jnp.mean(x, axis=1)
    i8 = jax.lax.broadcasted_iota(jnp.int32, (8, 8), 0)
    j8 = jax.lax.broadcasted_iota(jnp.int32, (8, 8), 1)
    eye8 = (i8 == j8).astype(jnp.float32)[None, :, :, None]
    dd = jnp.concatenate(
        [jnp.sum(x[:, 8 * t:8 * t + 8, 8 * t:8 * t + 8, :] * eye8, axis=2)
         for t in range(_N // 8)], axis=1)

    F = jnp.concatenate([r, c, dd], axis=-1)
    S = jax.lax.dot_general(F.reshape(_GB * _N, 3 * _D), ws,
                            (((1,), (0,)), ((), ())),
                            preferred_element_type=jnp.float32)
    S = S.reshape(_GB, _N, 3 * _D)

    s = jnp.mean(dd, axis=1)
    g = jnp.mean(r, axis=1)
    Sg = jnp.concatenate([s, g], axis=-1)
    consts = jnp.sum(Sg[:, :, None] * wg[None, :, :], axis=1)

    x3 = x.reshape(_GB * _N * _N, _D)
    YZ = jnp.dot(x3, wb, preferred_element_type=jnp.float32)
    Y = YZ[:, :_D].reshape(_GB, _N, _N, _D)
    Zb = YZ[:, _D:].astype(jnp.bfloat16).reshape(_GB, _N, _N, _D)
    acc = Y + jnp.swapaxes(Zb, 1, 2).astype(jnp.float32)
    acc += S[:, :, None, :_D]
    colk = S[:, :, _D:2 * _D] + consts[:, None, :_D]
    acc += colk[:, None, :, :]
    out_ref[...] = jnp.maximum(acc, 0.0)
    diag = S[..., 2 * _D:] + consts[:, None, _D:]
    for t in range(_N // 8):
        sl = slice(8 * t, 8 * t + 8)
        blk = acc[:, sl, sl, :] + eye8 * diag[:, sl, None, :]
        out_ref[:, sl, sl, :] = jnp.maximum(blk, 0.0)


def kernel(A, X, Wlins, Wmlp):
    del A
    out = pl.pallas_call(
        _main_kernel,
        grid=(_B // _GB,),
        in_specs=[
            pl.BlockSpec((15, _D, _D), lambda b: (0, 0, 0)),
            pl.BlockSpec((_D, _D), lambda b: (0, 0)),
            pl.BlockSpec((_GB, _N, _N, _D), lambda b: (b, 0, 0, 0)),
        ],
        out_specs=pl.BlockSpec((_GB, _N, _N, _D), lambda b: (b, 0, 0, 0)),
        out_shape=jax.ShapeDtypeStruct((_B, _N, _N, _D), jnp.float32),
        scratch_shapes=[
            pltpu.VMEM((_D, 2 * _D), jnp.float32),
            pltpu.VMEM((3 * _D, 3 * _D), jnp.float32),
            pltpu.VMEM((2 * _D, 2 * _D), jnp.float32),
        ],
    )(Wlins, Wmlp, X)
    return out
